# baseline (speedup 1.0000x reference)
import functools

import jax
import jax.numpy as jnp
from jax import lax
from jax.experimental import pallas as pl
from jax.experimental.pallas import tpu as pltpu
from jax.experimental.pallas import tpu_sc as plsc

BATCH = 16384
INPUT_DIM = 1000000
L = 26
NC = 2
NS = 16
NW = NC * NS
RPT = BATCH // NW
IPT = RPT * L
PRE = 6


def _sc_kernel(x_hbm, w_hbm, b_hbm, out_hbm, x_v, vals_v, acc_v, b_v,
               out_v, sem):
    wid = lax.axis_index("s") * NC + lax.axis_index("c")

    pltpu.sync_copy(x_hbm.at[wid], x_v)
    pltpu.sync_copy(b_hbm, b_v)

    w_flat = w_hbm.at[0]

    def gather_col(j):
        return pltpu.async_copy(
            w_flat.at[x_v.at[pl.ds(j * RPT, RPT)]],
            vals_v.at[pl.ds(j * RPT, RPT)],
            sem,
        )

    def add_col(j):
        def group_body(g, _):
            base = g * 16
            acc_v[pl.ds(base, 16)] = (
                acc_v[pl.ds(base, 16)] + vals_v[pl.ds(j * RPT + base, 16)]
            )
            return 0

        lax.fori_loop(0, RPT // 16, group_body, 0)

    def init_acc(g, _):
        acc_v[pl.ds(g * 16, 16)] = jnp.zeros((16,), jnp.float32)
        return 0

    lax.fori_loop(0, RPT // 16, init_acc, 0)

    copies = []
    for j in range(L):
        copies.append(gather_col(j))
        if j >= PRE:
            copies[j - PRE].wait()
            add_col(j - PRE)
    for j in range(L - PRE, L):
        copies[j].wait()
        add_col(j)

    bias = b_v[...]

    def final_body(g, _):
        base = g * 16
        z = (acc_v[pl.ds(base, 16)] + bias) * 2.0
        out_v[pl.ds(base, 16)] = 1.0 / (1.0 + jnp.exp(-z))
        return 0

    lax.fori_loop(0, RPT // 16, final_body, 0)

    pltpu.sync_copy(out_v, out_hbm.at[pl.ds(wid * RPT, RPT)])


@jax.jit
def _run(x3, w_flat, b16):
    mesh = plsc.VectorSubcoreMesh(core_axis_name="c", subcore_axis_name="s")
    f = functools.partial(
        pl.kernel,
        mesh=mesh,
        out_type=jax.ShapeDtypeStruct((BATCH,), jnp.float32),
        scratch_types=[
            pltpu.VMEM((IPT,), jnp.int32),
            pltpu.VMEM((IPT,), jnp.float32),
            pltpu.VMEM((RPT,), jnp.float32),
            pltpu.VMEM((16,), jnp.float32),
            pltpu.VMEM((RPT,), jnp.float32),
            pltpu.SemaphoreType.DMA,
        ],
    )(_sc_kernel)
    return f(x3, w_flat, b16)


def kernel(x, w, b):
    x3 = x.reshape(NW, RPT, L).transpose(0, 2, 1).reshape(NW, IPT)
    w_flat = w.reshape(1, INPUT_DIM)
    b16 = jnp.broadcast_to(b, (16,))
    out = _run(x3, w_flat, b16)
    return out.reshape(BATCH, 1)


# speedup vs baseline: 1.1303x; 1.1303x over previous
import functools

import jax
import jax.numpy as jnp
from jax import lax
from jax.experimental import pallas as pl
from jax.experimental.pallas import tpu as pltpu
from jax.experimental.pallas import tpu_sc as plsc

BATCH = 16384
INPUT_DIM = 1000000
L = 26
NC = 2
NS = 16
NW = NC * NS
RPT = BATCH // NW
IPT = RPT * L


def _sc_kernel(xt_hbm, w_hbm, b_hbm, out_hbm, x_v, vals_v, b_v, out_v, sem):
    wid = lax.axis_index("s") * NC + lax.axis_index("c")

    handles = [
        pltpu.async_copy(
            xt_hbm.at[j].at[pl.ds(wid * RPT, RPT)],
            x_v.at[pl.ds(j * RPT, RPT)],
            sem,
        )
        for j in range(L)
    ]
    pltpu.sync_copy(b_hbm, b_v)
    for h in handles:
        h.wait()

    pltpu.async_copy(w_hbm.at[0].at[x_v], vals_v, sem).wait()

    bias = b_v[...]

    def group_body(g, _):
        base = g * 16
        acc = jnp.zeros((16,), jnp.float32)
        for j in range(L):
            acc = acc + vals_v[pl.ds(j * RPT + base, 16)]
        z = (acc + bias) * 2.0
        out_v[pl.ds(base, 16)] = 1.0 / (1.0 + jnp.exp(-z))
        return 0

    lax.fori_loop(0, RPT // 16, group_body, 0)

    pltpu.sync_copy(out_v, out_hbm.at[pl.ds(wid * RPT, RPT)])


@jax.jit
def _run(xt, w_flat, b16):
    mesh = plsc.VectorSubcoreMesh(core_axis_name="c", subcore_axis_name="s")
    f = functools.partial(
        pl.kernel,
        mesh=mesh,
        out_type=jax.ShapeDtypeStruct((BATCH,), jnp.float32),
        scratch_types=[
            pltpu.VMEM((IPT,), jnp.int32),
            pltpu.VMEM((IPT,), jnp.float32),
            pltpu.VMEM((16,), jnp.float32),
            pltpu.VMEM((RPT,), jnp.float32),
            pltpu.SemaphoreType.DMA,
        ],
    )(_sc_kernel)
    return f(xt, w_flat, b16)


def kernel(x, w, b):
    xt = x.T
    w_flat = w.reshape(1, INPUT_DIM)
    b16 = jnp.broadcast_to(b, (16,))
    out = _run(xt, w_flat, b16)
    return out.reshape(BATCH, 1)
